# Initial kernel scaffold; baseline (speedup 1.0000x reference)
import jax
import jax.numpy as jnp
from jax.experimental import pallas as pl


def kernel(features_batch, adj_mats_batch, W, att_src, att_dst, bias):
    raise NotImplementedError("write your pallas kernel here")


# speedup vs baseline: 1.6724x; 1.6724x over previous
import functools

import jax
import jax.numpy as jnp
from jax.experimental import pallas as pl

_B, _N, _D = 8, 1024, 128
_H, _HD = 8, 16
_OUT = _H * _HD
_BI = 256
_NI = _N // _BI
_SLOPE = 0.2


def _gat_block_kernel(feat_ref, featb_ref, adj_ref, w_ref, asrc_ref, adst_ref,
                      bias_ref, out_ref):
    i_blk = pl.program_id(1)

    x = feat_ref[0]
    xp = jnp.dot(x, w_ref[...], preferred_element_type=jnp.float32)
    a_src = jnp.dot(xp, asrc_ref[...], preferred_element_type=jnp.float32)
    msrc = jnp.max(a_src, axis=0, keepdims=True)

    adj = adj_ref[0]
    jj = jax.lax.broadcasted_iota(jnp.int32, (_N, _BI), 0)
    ii = jax.lax.broadcasted_iota(jnp.int32, (_N, _BI), 1) + i_blk * _BI
    mask = jnp.maximum(adj, (jj == ii).astype(jnp.float32))

    x_blk = featb_ref[0]
    xp_blk = jnp.dot(x_blk, w_ref[...], preferred_element_type=jnp.float32)
    adst_blk = jnp.dot(xp_blk, adst_ref[...],
                       preferred_element_type=jnp.float32)
    adst_t = adst_blk.T
    ones = jnp.ones((_N, 1), dtype=jnp.float32)

    outs = []
    for h in range(_H):
        asrc_h = a_src[:, h:h + 1]
        adst_row = adst_t[h:h + 1, :]
        t = adst_row + msrc[:, h:h + 1]
        c_row = jnp.maximum(t, _SLOPE * t)
        s = asrc_h + adst_row
        e = jnp.maximum(s, _SLOPE * s)
        p = jnp.exp(e - c_row) * mask
        rhs = jnp.concatenate([xp[:, h * _HD:(h + 1) * _HD], ones], axis=1)
        nd = jax.lax.dot_general(p, rhs, (((0,), (0,)), ((), ())),
                                 preferred_element_type=jnp.float32)
        outs.append(nd[:, :_HD] / nd[:, _HD:_HD + 1])

    o = jnp.concatenate(outs, axis=1) + bias_ref[...]
    out_ref[0] = jnp.where(o > 0.0, o, jnp.exp(o) - 1.0)


@jax.jit
def kernel(features_batch, adj_mats_batch, W, att_src, att_dst, bias):
    eye = jnp.eye(_H, dtype=jnp.float32)
    a_src_mat = (att_src[:, :, None] * eye[:, None, :]).reshape(_OUT, _H)
    a_dst_mat = (att_dst[:, :, None] * eye[:, None, :]).reshape(_OUT, _H)
    bias2d = bias.reshape(1, _OUT)

    return pl.pallas_call(
        _gat_block_kernel,
        grid=(_B, _NI),
        in_specs=[
            pl.BlockSpec((1, _N, _D), lambda b, i: (b, 0, 0)),
            pl.BlockSpec((1, _BI, _D), lambda b, i: (b, i, 0)),
            pl.BlockSpec((1, _N, _BI), lambda b, i: (b, 0, i)),
            pl.BlockSpec((_D, _OUT), lambda b, i: (0, 0)),
            pl.BlockSpec((_OUT, _H), lambda b, i: (0, 0)),
            pl.BlockSpec((_OUT, _H), lambda b, i: (0, 0)),
            pl.BlockSpec((1, _OUT), lambda b, i: (0, 0)),
        ],
        out_specs=pl.BlockSpec((1, _BI, _OUT), lambda b, i: (b, i, 0)),
        out_shape=jax.ShapeDtypeStruct((_B, _N, _OUT), jnp.float32),
    )(features_batch, features_batch, adj_mats_batch, W, a_src_mat, a_dst_mat,
      bias2d)


# speedup vs baseline: 1.8767x; 1.1222x over previous
import functools

import jax
import jax.numpy as jnp
from jax.experimental import pallas as pl

_B, _N, _D = 8, 1024, 128
_H, _HD = 8, 16
_OUT = _H * _HD
_BI = 256
_NI = _N // _BI
_SLOPE = 0.2


def _gat_block_kernel(feat_ref, featb_ref, adj_ref, w_ref, asrc_ref, adst_ref,
                      bias_ref, out_ref):
    i_blk = pl.program_id(1)

    x = feat_ref[0]
    xp = jnp.dot(x, w_ref[...], preferred_element_type=jnp.float32)
    a_src = jnp.dot(xp, asrc_ref[...], preferred_element_type=jnp.float32)
    msrc = jnp.max(a_src, axis=0, keepdims=True)

    adj = adj_ref[0]
    jj = jax.lax.broadcasted_iota(jnp.int32, (_N, _BI), 0)
    ii = jax.lax.broadcasted_iota(jnp.int32, (_N, _BI), 1) + i_blk * _BI
    mask = jnp.maximum(adj, (jj == ii).astype(jnp.float32))

    x_blk = featb_ref[0]
    xp_blk = jnp.dot(x_blk, w_ref[...], preferred_element_type=jnp.float32)
    adst_blk = jnp.dot(xp_blk, adst_ref[...],
                       preferred_element_type=jnp.float32)
    adst_t = adst_blk.T
    ones = jnp.ones((_N, 1), dtype=jnp.float32)

    e_src1 = jnp.exp(a_src)
    e_src2 = jnp.exp(_SLOPE * a_src)
    t = adst_t + msrc.T
    c_rows = jnp.maximum(t, _SLOPE * t)
    f_dst1 = jnp.exp(adst_t - c_rows)
    f_dst2 = jnp.exp(_SLOPE * adst_t - c_rows)

    outs = []
    for h in range(_H):
        q1 = e_src1[:, h:h + 1] * f_dst1[h:h + 1, :]
        q2 = e_src2[:, h:h + 1] * f_dst2[h:h + 1, :]
        p = jnp.maximum(q1, q2) * mask
        rhs = jnp.concatenate([xp[:, h * _HD:(h + 1) * _HD], ones], axis=1)
        nd = jax.lax.dot_general(p, rhs, (((0,), (0,)), ((), ())),
                                 preferred_element_type=jnp.float32)
        outs.append(nd[:, :_HD] / nd[:, _HD:_HD + 1])

    o = jnp.concatenate(outs, axis=1) + bias_ref[...]
    out_ref[0] = jnp.where(o > 0.0, o, jnp.exp(o) - 1.0)


@jax.jit
def kernel(features_batch, adj_mats_batch, W, att_src, att_dst, bias):
    eye = jnp.eye(_H, dtype=jnp.float32)
    a_src_mat = (att_src[:, :, None] * eye[:, None, :]).reshape(_OUT, _H)
    a_dst_mat = (att_dst[:, :, None] * eye[:, None, :]).reshape(_OUT, _H)
    bias2d = bias.reshape(1, _OUT)

    return pl.pallas_call(
        _gat_block_kernel,
        grid=(_B, _NI),
        in_specs=[
            pl.BlockSpec((1, _N, _D), lambda b, i: (b, 0, 0)),
            pl.BlockSpec((1, _BI, _D), lambda b, i: (b, i, 0)),
            pl.BlockSpec((1, _N, _BI), lambda b, i: (b, 0, i)),
            pl.BlockSpec((_D, _OUT), lambda b, i: (0, 0)),
            pl.BlockSpec((_OUT, _H), lambda b, i: (0, 0)),
            pl.BlockSpec((_OUT, _H), lambda b, i: (0, 0)),
            pl.BlockSpec((1, _OUT), lambda b, i: (0, 0)),
        ],
        out_specs=pl.BlockSpec((1, _BI, _OUT), lambda b, i: (b, i, 0)),
        out_shape=jax.ShapeDtypeStruct((_B, _N, _OUT), jnp.float32),
    )(features_batch, features_batch, adj_mats_batch, W, a_src_mat, a_dst_mat,
      bias2d)


# speedup vs baseline: 1.8892x; 1.0066x over previous
import functools

import jax
import jax.numpy as jnp
from jax.experimental import pallas as pl

_B, _N, _D = 8, 1024, 128
_H, _HD = 8, 16
_OUT = _H * _HD
_BI = 256
_NI = _N // _BI
_SLOPE = 0.2


def _gat_block_kernel(feat_ref, featb_ref, adj_ref, w_ref, asrc_ref, adst_ref,
                      bias_ref, out_ref):
    i_blk = pl.program_id(1)

    x = feat_ref[0]
    xp = jnp.dot(x, w_ref[...], preferred_element_type=jnp.float32)
    a_src = jnp.dot(xp, asrc_ref[...], preferred_element_type=jnp.float32)
    msrc = jnp.max(a_src, axis=0, keepdims=True)

    adj = adj_ref[0]
    jj = jax.lax.broadcasted_iota(jnp.int32, (_N, _BI), 0)
    ii = jax.lax.broadcasted_iota(jnp.int32, (_N, _BI), 1) + i_blk * _BI
    mask = jnp.maximum(adj, (jj == ii).astype(jnp.float32))

    x_blk = featb_ref[0]
    xp_blk = jnp.dot(x_blk, w_ref[...], preferred_element_type=jnp.float32)
    adst_blk = jnp.dot(xp_blk, adst_ref[...],
                       preferred_element_type=jnp.float32)
    adst_t = adst_blk.T
    ones = jnp.ones((_N, 1), dtype=jnp.float32)

    e_src1 = jnp.exp(a_src)
    e_src2 = jnp.exp(_SLOPE * a_src)
    t = adst_t + msrc.T
    c_rows = jnp.maximum(t, _SLOPE * t)
    f_dst1 = jnp.exp(adst_t - c_rows)
    f_dst2 = jnp.exp(_SLOPE * adst_t - c_rows)

    zrow = jnp.zeros((1, _BI), dtype=jnp.float32)
    outs = []
    for h in range(_H):
        lhs2 = jnp.concatenate([e_src1[:, h:h + 1], e_src2[:, h:h + 1]], axis=1)
        rhs2 = jnp.concatenate([
            jnp.concatenate([f_dst1[h:h + 1, :], zrow], axis=1),
            jnp.concatenate([zrow, f_dst2[h:h + 1, :]], axis=1),
        ], axis=0)
        q12 = jnp.dot(lhs2, rhs2, preferred_element_type=jnp.float32)
        p = jnp.maximum(q12[:, :_BI], q12[:, _BI:]) * mask
        rhs = jnp.concatenate([xp[:, h * _HD:(h + 1) * _HD], ones], axis=1)
        nd = jax.lax.dot_general(p, rhs, (((0,), (0,)), ((), ())),
                                 preferred_element_type=jnp.float32)
        outs.append(nd[:, :_HD] / nd[:, _HD:_HD + 1])

    o = jnp.concatenate(outs, axis=1) + bias_ref[...]
    out_ref[0] = jnp.where(o > 0.0, o, jnp.exp(o) - 1.0)


@jax.jit
def kernel(features_batch, adj_mats_batch, W, att_src, att_dst, bias):
    eye = jnp.eye(_H, dtype=jnp.float32)
    a_src_mat = (att_src[:, :, None] * eye[:, None, :]).reshape(_OUT, _H)
    a_dst_mat = (att_dst[:, :, None] * eye[:, None, :]).reshape(_OUT, _H)
    bias2d = bias.reshape(1, _OUT)

    return pl.pallas_call(
        _gat_block_kernel,
        grid=(_B, _NI),
        in_specs=[
            pl.BlockSpec((1, _N, _D), lambda b, i: (b, 0, 0)),
            pl.BlockSpec((1, _BI, _D), lambda b, i: (b, i, 0)),
            pl.BlockSpec((1, _N, _BI), lambda b, i: (b, 0, i)),
            pl.BlockSpec((_D, _OUT), lambda b, i: (0, 0)),
            pl.BlockSpec((_OUT, _H), lambda b, i: (0, 0)),
            pl.BlockSpec((_OUT, _H), lambda b, i: (0, 0)),
            pl.BlockSpec((1, _OUT), lambda b, i: (0, 0)),
        ],
        out_specs=pl.BlockSpec((1, _BI, _OUT), lambda b, i: (b, i, 0)),
        out_shape=jax.ShapeDtypeStruct((_B, _N, _OUT), jnp.float32),
    )(features_batch, features_batch, adj_mats_batch, W, a_src_mat, a_dst_mat,
      bias2d)
